# Initial kernel scaffold; baseline (speedup 1.0000x reference)
import jax
import jax.numpy as jnp
from jax.experimental import pallas as pl


def kernel(query, Wq, keys_left, keys_right, values, Wo, gamma, beta):
    raise NotImplementedError("write your pallas kernel here")


# speedup vs baseline: 5.8916x; 5.8916x over previous
import functools

import jax
import jax.numpy as jnp
from jax import lax
from jax.experimental import pallas as pl
from jax.experimental.pallas import tpu as pltpu
from jax.experimental.pallas import tpu_sc as plsc

D_MODEL = 2048
D_MEMORY = 512
NUM_KEYS = 256
TOP_K = 8
NUM_HEADS = 8
D_HEAD = D_MEMORY // NUM_HEADS
NEG = -1e30


_BLK1 = 512


def _tc1_body(x_ref, wq_ref, kl_ref, kr_ref, slots_ref, wts_ref):
    hi = lax.Precision.DEFAULT
    x = x_ref[...]
    q = lax.dot_general(x, wq_ref[...], (((1,), (1,)), ((), ())),
                        precision=hi,
                        preferred_element_type=jnp.float32)

    def norm_keys(k):
        n = jnp.sqrt(jnp.sum(k * k, axis=1, keepdims=True))
        return k / jnp.maximum(n, 1e-12)

    kln = norm_keys(kl_ref[...])
    krn = norm_keys(kr_ref[...])

    iota_k = lax.broadcasted_iota(jnp.int32, (_BLK1, NUM_KEYS), 1)

    def topk8(s):
        vals, idxs = [], []
        for _ in range(TOP_K):
            m = jnp.max(s, axis=1, keepdims=True)
            eq = s == m
            idx = jnp.min(jnp.where(eq, iota_k, NUM_KEYS), axis=1,
                          keepdims=True)
            vals.append(m)
            idxs.append(idx)
            s = jnp.where(iota_k == idx, NEG, s)
        return jnp.concatenate(vals, axis=1), jnp.concatenate(idxs, axis=1)

    for h in range(NUM_HEADS):
        qh = q[:, h * D_HEAD:(h + 1) * D_HEAD]
        nh = jnp.sqrt(jnp.sum(qh * qh, axis=1, keepdims=True))
        qh = qh / jnp.maximum(nh, 1e-12)
        sl = lax.dot_general(qh, kln[h * NUM_KEYS:(h + 1) * NUM_KEYS, :],
                             (((1,), (1,)), ((), ())), precision=hi,
                             preferred_element_type=jnp.float32)
        sr = lax.dot_general(qh, krn[h * NUM_KEYS:(h + 1) * NUM_KEYS, :],
                             (((1,), (1,)), ((), ())), precision=hi,
                             preferred_element_type=jnp.float32)
        ls, li = topk8(sl)
        rs, ri = topk8(sr)
        u = jnp.exp(ls - ls[:, 0:1])
        v = jnp.exp(rs - rs[:, 0:1])
        zinv = 1.0 / (jnp.sum(u, axis=1, keepdims=True)
                      * jnp.sum(v, axis=1, keepdims=True))
        w = jnp.concatenate([u[:, a:a + 1] * v for a in range(TOP_K)],
                            axis=1) * zinv
        slots = jnp.concatenate(
            [li[:, a:a + 1] * NUM_KEYS + ri for a in range(TOP_K)],
            axis=1) + h * (NUM_KEYS * NUM_KEYS)
        wts_ref[:, h * 64:(h + 1) * 64] = w
        slots_ref[:, h * 64:(h + 1) * 64] = slots


def _run_tc1(x, wq, kl, kr):
    bt = x.shape[0]
    grid = (bt // _BLK1,)
    return pl.pallas_call(
        _tc1_body,
        grid=grid,
        in_specs=[
            pl.BlockSpec((_BLK1, D_MODEL), lambda i: (i, 0)),
            pl.BlockSpec((D_MEMORY, D_MODEL), lambda i: (0, 0)),
            pl.BlockSpec((NUM_HEADS * NUM_KEYS, D_HEAD), lambda i: (0, 0)),
            pl.BlockSpec((NUM_HEADS * NUM_KEYS, D_HEAD), lambda i: (0, 0)),
        ],
        out_specs=[
            pl.BlockSpec((_BLK1, D_MEMORY), lambda i: (i, 0)),
            pl.BlockSpec((_BLK1, D_MEMORY), lambda i: (i, 0)),
        ],
        out_shape=[
            jax.ShapeDtypeStruct((bt, D_MEMORY), jnp.int32),
            jax.ShapeDtypeStruct((bt, D_MEMORY), jnp.float32),
        ],
    )(x, wq, kl, kr)


_NC, _NS, _L = 2, 16, 16
_NW = _NC * _NS
_NBUF = 4
_CHUNK = 128


def _sc_retrieve(values, slots_flat, wts_flat, n_pairs):
    ppw = n_pairs // _NW
    n_chunks = ppw // _CHUNK
    mesh = plsc.VectorSubcoreMesh(core_axis_name="c", subcore_axis_name="s")

    def body(values_hbm, slots_hbm, wts_hbm, z_hbm,
             idx_v, wts_v, zbuf, r0, r1, r2, r3, s0, s1, s2, s3):
        rows = (r0, r1, r2, r3)
        sems = (s0, s1, s2, s3)
        wid = lax.axis_index("s") * _NC + lax.axis_index("c")
        base = wid * ppw

        pltpu.sync_copy(slots_hbm.at[pl.ds(base * 64, ppw * 64)], idx_v)

        def gather(pair_local, b):
            pltpu.async_copy(
                values_hbm.at[idx_v.at[pl.ds(pair_local * 64, 64)]],
                rows[b], sems[b])

        for b in range(_NBUF):
            gather(b, b)

        def chunk_body(c):
            cb = c * _CHUNK
            pltpu.sync_copy(
                wts_hbm.at[pl.ds((base + cb) * 64, _CHUNK * 64)], wts_v)

            def quad_body(qi):
                for b in range(_NBUF):
                    lo = qi * _NBUF + b
                    p = cb + lo
                    pltpu.make_async_copy(
                        values_hbm.at[idx_v.at[pl.ds(p * 64, 64)]],
                        rows[b], sems[b]).wait()
                    def jj_body(jj, acc):
                        w16 = wts_v[pl.ds(lo * 64 + jj * 16, 16)]
                        acc = list(acc)
                        for j16 in range(16):
                            w = jnp.take_along_axis(
                                w16, jnp.full((_L,), j16, jnp.int32), axis=0,
                                mode="promise_in_bounds")
                            j = jj * 16 + j16
                            for cc in range(4):
                                acc[cc] = acc[cc] + w * rows[b][j, pl.ds(cc * 16, 16)]
                        return tuple(acc)

                    acc = pl.loop(
                        0, 4,
                        init_carry=tuple(
                            jnp.zeros((_L,), jnp.float32) for _ in range(4)),
                    )(jj_body)
                    for cc in range(4):
                        zbuf[pl.ds(lo * 64 + cc * 16, 16)] = acc[cc]
                    nxt = p + _NBUF
                    @pl.when(nxt < ppw)
                    def _():
                        gather(nxt, b)
                return None

            pl.loop(0, _CHUNK // _NBUF)(quad_body)
            pltpu.sync_copy(
                zbuf, z_hbm.at[pl.ds((base + cb) * 64, _CHUNK * 64)])
            return None

        pl.loop(0, n_chunks)(chunk_body)

    f = pl.kernel(
        body,
        out_type=jax.ShapeDtypeStruct((n_pairs * 64,), jnp.float32),
        mesh=mesh,
        compiler_params=pltpu.CompilerParams(use_tc_tiling_on_sc=False),
        scratch_types=[
            pltpu.VMEM((ppw * 64,), jnp.int32),
            pltpu.VMEM((_CHUNK * 64,), jnp.float32),
            pltpu.VMEM((_CHUNK * 64,), jnp.float32),
        ] + [pltpu.VMEM((64, D_HEAD), jnp.float32) for _ in range(_NBUF)]
          + [pltpu.SemaphoreType.DMA for _ in range(_NBUF)],
    )
    return f(values, slots_flat, wts_flat)


_BLK2 = 512


def _tc2_body(z_ref, wo_ref, g_ref, b_ref, out_ref):
    zm = lax.dot_general(z_ref[...], wo_ref[...], (((1,), (1,)), ((), ())),
                         preferred_element_type=jnp.float32)
    mu = jnp.mean(zm, axis=1, keepdims=True)
    zc = zm - mu
    var = jnp.mean(zc * zc, axis=1, keepdims=True)
    out_ref[...] = zc * lax.rsqrt(var + 1e-5) * g_ref[...] + b_ref[...]


def _run_tc2(z, wo, gamma, beta):
    bt = z.shape[0]
    grid = (bt // _BLK2,)
    return pl.pallas_call(
        _tc2_body,
        grid=grid,
        in_specs=[
            pl.BlockSpec((_BLK2, D_MEMORY), lambda i: (i, 0)),
            pl.BlockSpec((D_MODEL, D_MEMORY), lambda i: (0, 0)),
            pl.BlockSpec((1, D_MODEL), lambda i: (0, 0)),
            pl.BlockSpec((1, D_MODEL), lambda i: (0, 0)),
        ],
        out_specs=pl.BlockSpec((_BLK2, D_MODEL), lambda i: (i, 0)),
        out_shape=jax.ShapeDtypeStruct((bt, D_MODEL), jnp.float32),
    )(z, wo, gamma.reshape(1, D_MODEL), beta.reshape(1, D_MODEL))


def kernel(query, Wq, keys_left, keys_right, values, Wo, gamma, beta):
    B, T, _ = query.shape
    bt = B * T
    x = query.reshape(bt, D_MODEL)
    kl = keys_left.reshape(NUM_HEADS * NUM_KEYS, D_HEAD)
    kr = keys_right.reshape(NUM_HEADS * NUM_KEYS, D_HEAD)

    slots, wts = _run_tc1(x, Wq, kl, kr)

    n_pairs = bt * NUM_HEADS
    z_flat = _sc_retrieve(values, slots.reshape(-1), wts.reshape(-1), n_pairs)
    z = z_flat.reshape(bt, D_MEMORY)

    out = _run_tc2(z, Wo, gamma, beta)
    return out.reshape(B, T, D_MODEL)


# speedup vs baseline: 5.9719x; 1.0136x over previous
import functools

import jax
import jax.numpy as jnp
from jax import lax
from jax.experimental import pallas as pl
from jax.experimental.pallas import tpu as pltpu
from jax.experimental.pallas import tpu_sc as plsc

D_MODEL = 2048
D_MEMORY = 512
NUM_KEYS = 256
TOP_K = 8
NUM_HEADS = 8
D_HEAD = D_MEMORY // NUM_HEADS
NEG = -1e30


_BLK1 = 512


def _tc1_body(x_ref, wq_ref, kl_ref, kr_ref, slots_ref, wts_ref):
    hi = lax.Precision.DEFAULT
    x = x_ref[...]
    q = lax.dot_general(x, wq_ref[...], (((1,), (1,)), ((), ())),
                        precision=hi,
                        preferred_element_type=jnp.float32)

    def norm_keys(k):
        n = jnp.sqrt(jnp.sum(k * k, axis=1, keepdims=True))
        return k / jnp.maximum(n, 1e-12)

    kln = norm_keys(kl_ref[...])
    krn = norm_keys(kr_ref[...])

    iota_k = lax.broadcasted_iota(jnp.int32, (_BLK1, NUM_KEYS), 1)

    def topk8(s):
        vals, idxs = [], []
        for _ in range(TOP_K):
            m = jnp.max(s, axis=1, keepdims=True)
            eq = s == m
            idx = jnp.min(jnp.where(eq, iota_k, NUM_KEYS), axis=1,
                          keepdims=True)
            vals.append(m)
            idxs.append(idx)
            s = jnp.where(iota_k == idx, NEG, s)
        return jnp.concatenate(vals, axis=1), jnp.concatenate(idxs, axis=1)

    for h in range(NUM_HEADS):
        qh = q[:, h * D_HEAD:(h + 1) * D_HEAD]
        nh = jnp.sqrt(jnp.sum(qh * qh, axis=1, keepdims=True))
        qh = qh / jnp.maximum(nh, 1e-12)
        sl = lax.dot_general(qh, kln[h * NUM_KEYS:(h + 1) * NUM_KEYS, :],
                             (((1,), (1,)), ((), ())), precision=hi,
                             preferred_element_type=jnp.float32)
        sr = lax.dot_general(qh, krn[h * NUM_KEYS:(h + 1) * NUM_KEYS, :],
                             (((1,), (1,)), ((), ())), precision=hi,
                             preferred_element_type=jnp.float32)
        ls, li = topk8(sl)
        rs, ri = topk8(sr)
        u = jnp.exp(ls - ls[:, 0:1])
        v = jnp.exp(rs - rs[:, 0:1])
        zinv = 1.0 / (jnp.sum(u, axis=1, keepdims=True)
                      * jnp.sum(v, axis=1, keepdims=True))
        w = jnp.concatenate([u[:, a:a + 1] * v for a in range(TOP_K)],
                            axis=1) * zinv
        slots = jnp.concatenate(
            [li[:, a:a + 1] * NUM_KEYS + ri for a in range(TOP_K)],
            axis=1) + h * (NUM_KEYS * NUM_KEYS)
        wts_ref[:, h * 64:(h + 1) * 64] = w
        slots_ref[:, h * 64:(h + 1) * 64] = slots


def _run_tc1(x, wq, kl, kr):
    bt = x.shape[0]
    grid = (bt // _BLK1,)
    return pl.pallas_call(
        _tc1_body,
        grid=grid,
        in_specs=[
            pl.BlockSpec((_BLK1, D_MODEL), lambda i: (i, 0)),
            pl.BlockSpec((D_MEMORY, D_MODEL), lambda i: (0, 0)),
            pl.BlockSpec((NUM_HEADS * NUM_KEYS, D_HEAD), lambda i: (0, 0)),
            pl.BlockSpec((NUM_HEADS * NUM_KEYS, D_HEAD), lambda i: (0, 0)),
        ],
        out_specs=[
            pl.BlockSpec((_BLK1, D_MEMORY), lambda i: (i, 0)),
            pl.BlockSpec((_BLK1, D_MEMORY), lambda i: (i, 0)),
        ],
        out_shape=[
            jax.ShapeDtypeStruct((bt, D_MEMORY), jnp.int32),
            jax.ShapeDtypeStruct((bt, D_MEMORY), jnp.float32),
        ],
    )(x, wq, kl, kr)


_NC, _NS, _L = 2, 16, 16
_NW = _NC * _NS
_NBUF = 4
_CHUNK = 128


def _sc_retrieve(values, slots_flat, wts_flat, n_pairs):
    ppw = n_pairs // _NW
    n_chunks = ppw // _CHUNK
    mesh = plsc.VectorSubcoreMesh(core_axis_name="c", subcore_axis_name="s")

    def body(values_hbm, slots_hbm, wts_hbm, z_hbm,
             idx_v, wts_v, zbuf, r0, r1, r2, r3, s0, s1, s2, s3):
        rows = (r0, r1, r2, r3)
        sems = (s0, s1, s2, s3)
        wid = lax.axis_index("s") * _NC + lax.axis_index("c")
        base = wid * ppw

        pltpu.sync_copy(slots_hbm.at[pl.ds(base * 64, ppw * 64)], idx_v)

        def gather(pair_local, b):
            pltpu.async_copy(
                values_hbm.at[idx_v.at[pl.ds(pair_local * 64, 64)]],
                rows[b], sems[b])

        for b in range(_NBUF):
            gather(b, b)

        def chunk_body(c):
            cb = c * _CHUNK
            pltpu.sync_copy(
                wts_hbm.at[pl.ds((base + cb) * 64, _CHUNK * 64)], wts_v)

            def quad_body(qi):
                for b in range(_NBUF):
                    lo = qi * _NBUF + b
                    p = cb + lo
                    pltpu.make_async_copy(
                        values_hbm.at[idx_v.at[pl.ds(p * 64, 64)]],
                        rows[b], sems[b]).wait()
                    def jj_body(jj, acc):
                        w16 = wts_v[pl.ds(lo * 64 + jj * 16, 16)]
                        acc = list(acc)
                        for j16 in range(16):
                            w = jnp.take_along_axis(
                                w16, jnp.full((_L,), j16, jnp.int32), axis=0,
                                mode="promise_in_bounds")
                            j = jj * 16 + j16
                            for cc in range(4):
                                acc[cc] = acc[cc] + w * rows[b][j, pl.ds(cc * 16, 16)]
                        return tuple(acc)

                    acc = pl.loop(
                        0, 4,
                        init_carry=tuple(
                            jnp.zeros((_L,), jnp.float32) for _ in range(4)),
                    )(jj_body)
                    for cc in range(4):
                        zbuf[pl.ds(lo * 64 + cc * 16, 16)] = acc[cc]
                    nxt = p + _NBUF
                    @pl.when(nxt < ppw)
                    def _():
                        gather(nxt, b)
                return None

            pl.loop(0, _CHUNK // _NBUF)(quad_body)
            pltpu.sync_copy(
                zbuf, z_hbm.at[pl.ds((base + cb) * 64, _CHUNK * 64)])
            return None

        pl.loop(0, n_chunks)(chunk_body)

    f = pl.kernel(
        body,
        out_type=jax.ShapeDtypeStruct((n_pairs * 64,), jnp.float32),
        mesh=mesh,
        compiler_params=pltpu.CompilerParams(use_tc_tiling_on_sc=False),
        scratch_types=[
            pltpu.VMEM((ppw * 64,), jnp.int32),
            pltpu.VMEM((_CHUNK * 64,), jnp.float32),
            pltpu.VMEM((_CHUNK * 64,), jnp.float32),
        ] + [pltpu.VMEM((64, D_HEAD), jnp.float32) for _ in range(_NBUF)]
          + [pltpu.SemaphoreType.DMA for _ in range(_NBUF)],
    )
    return f(values, slots_flat, wts_flat)


_BLK2 = 512


def _tc2_body(z_ref, wo_ref, g_ref, b_ref, out_ref):
    zm = lax.dot_general(z_ref[...], wo_ref[...], (((1,), (1,)), ((), ())),
                         preferred_element_type=jnp.float32)
    mu = jnp.mean(zm, axis=1, keepdims=True)
    zc = zm - mu
    var = jnp.mean(zc * zc, axis=1, keepdims=True)
    out_ref[...] = zc * lax.rsqrt(var + 1e-5) * g_ref[...] + b_ref[...]


def _run_tc2(z, wo, gamma, beta):
    bt = z.shape[0]
    grid = (bt // _BLK2,)
    return pl.pallas_call(
        _tc2_body,
        grid=grid,
        in_specs=[
            pl.BlockSpec((_BLK2, D_MEMORY), lambda i: (i, 0)),
            pl.BlockSpec((D_MODEL, D_MEMORY), lambda i: (0, 0)),
            pl.BlockSpec((1, D_MODEL), lambda i: (0, 0)),
            pl.BlockSpec((1, D_MODEL), lambda i: (0, 0)),
        ],
        out_specs=pl.BlockSpec((_BLK2, D_MODEL), lambda i: (i, 0)),
        out_shape=jax.ShapeDtypeStruct((bt, D_MODEL), jnp.float32),
    )(z, wo, gamma.reshape(1, D_MODEL), beta.reshape(1, D_MODEL))


_N_CHUNKS = 2


def kernel(query, Wq, keys_left, keys_right, values, Wo, gamma, beta):
    B, T, _ = query.shape
    bt = B * T
    x = query.reshape(bt, D_MODEL)
    kl = keys_left.reshape(NUM_HEADS * NUM_KEYS, D_HEAD)
    kr = keys_right.reshape(NUM_HEADS * NUM_KEYS, D_HEAD)

    bt_c = bt // _N_CHUNKS
    outs = []
    for c in range(_N_CHUNKS):
        xc = lax.slice_in_dim(x, c * bt_c, (c + 1) * bt_c, axis=0)
        slots, wts = _run_tc1(xc, Wq, kl, kr)
        n_pairs = bt_c * NUM_HEADS
        z_flat = _sc_retrieve(values, slots.reshape(-1), wts.reshape(-1),
                              n_pairs)
        z = z_flat.reshape(bt_c, D_MEMORY)
        outs.append(_run_tc2(z, Wo, gamma, beta))
    out = jnp.concatenate(outs, axis=0)
    return out.reshape(B, T, D_MODEL)
